# baseline (speedup 1.0000x reference)
import jax
import jax.numpy as jnp
from jax.experimental import pallas as pl

NUM_CODEBOOKS = 4
CODEBOOK_SIZE = 1024
EMBED_DIM = 256
BATCH = 16384

TILE_B = 1024


def _rqvae_body(x_ref, cb_hi_ref, cb_mid_ref, cb_lo_ref, c2_ref,
                recon_ref, codes_ref, loss_ref):
    x = x_ref[...]
    recon = jnp.zeros_like(x)
    residual = x
    loss_part = jnp.zeros((), dtype=jnp.float32)
    codes_list = []
    for i in range(NUM_CODEBOOKS):
        c_hi = cb_hi_ref[i]
        c2 = c2_ref[i][:1, :]
        r2 = jnp.sum(residual * residual, axis=-1, keepdims=True)
        dot = jax.lax.dot_general(
            residual.astype(jnp.bfloat16), c_hi,
            (((1,), (1,)), ((), ())),
            preferred_element_type=jnp.float32)
        dist = (r2 + c2) - 2.0 * dot
        mn = jnp.min(dist, axis=-1, keepdims=True)
        iota = jax.lax.broadcasted_iota(jnp.int32, dist.shape, 1)
        code = jnp.min(jnp.where(dist == mn, iota, CODEBOOK_SIZE), axis=-1)

        onehot = (iota == code[:, None]).astype(jnp.bfloat16)
        dg = lambda a, b: jax.lax.dot_general(
            a, b, (((1,), (0,)), ((), ())), preferred_element_type=jnp.float32)
        q = ((dg(onehot, c_hi) + dg(onehot, cb_mid_ref[i]))
             + dg(onehot, cb_lo_ref[i]))

        diff = q - residual
        loss_part = loss_part + jnp.sum(diff * diff)
        recon = recon + q
        residual = x - recon
        codes_list.append(code)

    recon_ref[...] = recon
    codes_ref[...] = jnp.stack(codes_list, axis=-1)
    @pl.when(pl.program_id(0) == 0)
    def _init():
        loss_ref[...] = jnp.zeros_like(loss_ref)
    loss_ref[...] += loss_part[None, None]


@jax.jit
def kernel(x, codebooks):
    num_tiles = BATCH // TILE_B
    cb_hi = codebooks.astype(jnp.bfloat16)
    rem = codebooks - cb_hi.astype(jnp.float32)
    cb_mid = rem.astype(jnp.bfloat16)
    cb_lo = (rem - cb_mid.astype(jnp.float32)).astype(jnp.bfloat16)
    c2 = jnp.sum(codebooks ** 2, axis=-1)
    c2b = jnp.broadcast_to(c2[:, None, :], (NUM_CODEBOOKS, 8, CODEBOOK_SIZE))

    recon, codes, loss_sum = pl.pallas_call(
        _rqvae_body,
        grid=(num_tiles,),
        in_specs=[
            pl.BlockSpec((TILE_B, EMBED_DIM), lambda b: (b, 0)),
            pl.BlockSpec((NUM_CODEBOOKS, CODEBOOK_SIZE, EMBED_DIM),
                         lambda b: (0, 0, 0)),
            pl.BlockSpec((NUM_CODEBOOKS, CODEBOOK_SIZE, EMBED_DIM),
                         lambda b: (0, 0, 0)),
            pl.BlockSpec((NUM_CODEBOOKS, CODEBOOK_SIZE, EMBED_DIM),
                         lambda b: (0, 0, 0)),
            pl.BlockSpec((NUM_CODEBOOKS, 8, CODEBOOK_SIZE),
                         lambda b: (0, 0, 0)),
        ],
        out_specs=[
            pl.BlockSpec((TILE_B, EMBED_DIM), lambda b: (b, 0)),
            pl.BlockSpec((TILE_B, NUM_CODEBOOKS), lambda b: (b, 0)),
            pl.BlockSpec((1, 1), lambda b: (0, 0)),
        ],
        out_shape=[
            jax.ShapeDtypeStruct((BATCH, EMBED_DIM), jnp.float32),
            jax.ShapeDtypeStruct((BATCH, NUM_CODEBOOKS), jnp.int32),
            jax.ShapeDtypeStruct((1, 1), jnp.float32),
        ],
    )(x, cb_hi, cb_mid, cb_lo, c2b)
    total_loss = (loss_sum[0, 0] * 2.0) / (BATCH * EMBED_DIM)
    return recon, codes, total_loss
